# Initial kernel scaffold; baseline (speedup 1.0000x reference)
import jax
import jax.numpy as jnp
from jax.experimental import pallas as pl


def kernel(X, pos, dist, W_D1, b_D1, W_D2, b_D2, W_F1, b_F1, W_F2, b_F2):
    raise NotImplementedError("write your pallas kernel here")


# speedup vs baseline: 4.4293x; 4.4293x over previous
import jax
import jax.numpy as jnp
from jax.experimental import pallas as pl
from jax.experimental.pallas import tpu as pltpu

B, C, N, T, K, OUT = 2, 32, 128, 24, 8, 64
H = OUT // 2
NT = N * T
TH = T * H
TC = T * C
BIG = 3.0e38


def _topk_onehots(dmat, k):
    cols = jax.lax.broadcasted_iota(jnp.int32, (N, N), 1)
    ohs = []
    d = dmat
    for _ in range(k):
        m = jnp.min(d, axis=1, keepdims=True)
        idx = jnp.min(jnp.where(d == m, cols, N), axis=1, keepdims=True)
        sel = cols == idx
        ohs.append(sel.astype(jnp.float32))
        d = jnp.where(sel, BIG, d)
    return ohs


def _body_a(xg_ref, pos2_ref, dist_ref,
            wfa_ref, wfb_ref, wdx_ref, wdxb_ref, wdpa_ref, wdpb_ref,
            bf1_ref, bd1_ref,
            hf_ref, hd_ref):
    hi = jax.lax.Precision.HIGHEST
    xg = xg_ref[0]
    pos2 = pos2_ref[...]
    dist = dist_ref[...]

    rn = jnp.sum(xg * xg, axis=1, keepdims=True)
    g = jax.lax.dot_general(xg, xg, (((1,), (1,)), ((), ())), precision=hi)
    d = rn + jnp.transpose(rn) - 2.0 * g

    oh_f = _topk_onehots(d, K)
    oh_d = _topk_onehots(dist, K)

    a_f = jnp.dot(xg, wfa_ref[...], precision=hi) + bf1_ref[...]
    b_f = jnp.dot(xg, wfb_ref[...], precision=hi)
    a_d = (jnp.dot(xg, wdx_ref[...], precision=hi)
           + jnp.dot(pos2, wdpa_ref[...], precision=hi) + bd1_ref[...])
    b_d = (jnp.dot(xg, wdxb_ref[...], precision=hi)
           + jnp.dot(pos2, wdpb_ref[...], precision=hi))

    for k in range(K):
        hf_ref[0, k] = jax.nn.relu(
            a_f + jnp.dot(oh_f[k], b_f, precision=hi))
        hd_ref[0, k] = jax.nn.relu(
            a_d + jnp.dot(oh_d[k], b_d, precision=hi))


def _body_b(hf_ref, hd_ref, wd2t_ref, bd2_ref, wf2t_ref, bf2_ref, out_ref,
            accf_ref, accd_ref):
    hi = jax.lax.Precision.HIGHEST
    k = pl.program_id(1)
    o_f = jax.nn.relu(
        jnp.dot(hf_ref[0, 0], wf2t_ref[...], precision=hi) + bf2_ref[...])
    o_d = jax.nn.relu(
        jnp.dot(hd_ref[0, 0], wd2t_ref[...], precision=hi) + bd2_ref[...])

    @pl.when(k == 0)
    def _():
        accf_ref[...] = o_f
        accd_ref[...] = o_d

    @pl.when(k > 0)
    def _():
        accf_ref[...] = jnp.maximum(accf_ref[...], o_f)
        accd_ref[...] = jnp.maximum(accd_ref[...], o_d)

    @pl.when(k == K - 1)
    def _():
        out_ref[0] = accf_ref[...] + accd_ref[...]


def kernel(X, pos, dist, W_D1, b_D1, W_D2, b_D2, W_F1, b_F1, W_F2, b_F2):
    xg = jnp.reshape(jnp.transpose(X, (0, 2, 3, 1)), (B, N, TC))
    pos2 = jnp.tile(jnp.transpose(pos), (1, T))

    wf1t = W_F1.T
    wa, wb = wf1t[:C], wf1t[C:]
    wd1t = W_D1.T
    wx, wp = wd1t[:C], wd1t[C:C + 2]
    wdx, wdp = wd1t[C + 2:2 * C + 2], wd1t[2 * C + 2:]

    eyeT = jnp.eye(T, dtype=jnp.float32)
    bd = lambda w: jnp.kron(eyeT, w)
    w_fa = bd(wa - wb)
    w_fb = bd(wb)
    w_dx = bd(wx - wdx)
    w_dxb = bd(wdx)
    w_dpa = bd(wp - wdp)
    w_dpb = bd(wdp)
    bf1 = jnp.tile(b_F1, T)[None, :]
    bd1 = jnp.tile(b_D1, T)[None, :]

    bs = pl.BlockSpec
    hf, hd = pl.pallas_call(
        _body_a,
        grid=(B,),
        in_specs=[
            bs((1, N, TC), lambda b: (b, 0, 0)),
            bs((N, 2 * T), lambda b: (0, 0)),
            bs((N, N), lambda b: (0, 0)),
            bs((TC, TH), lambda b: (0, 0)),
            bs((TC, TH), lambda b: (0, 0)),
            bs((TC, TH), lambda b: (0, 0)),
            bs((TC, TH), lambda b: (0, 0)),
            bs((2 * T, TH), lambda b: (0, 0)),
            bs((2 * T, TH), lambda b: (0, 0)),
            bs((1, TH), lambda b: (0, 0)),
            bs((1, TH), lambda b: (0, 0)),
        ],
        out_specs=[
            bs((1, K, N, TH), lambda b: (b, 0, 0, 0)),
            bs((1, K, N, TH), lambda b: (b, 0, 0, 0)),
        ],
        out_shape=[
            jax.ShapeDtypeStruct((B, K, N, TH), jnp.float32),
            jax.ShapeDtypeStruct((B, K, N, TH), jnp.float32),
        ],
    )(xg, pos2, dist, w_fa, w_fb, w_dx, w_dxb, w_dpa, w_dpb, bf1, bd1)

    hf = jnp.reshape(hf, (B, K, NT, H))
    hd = jnp.reshape(hd, (B, K, NT, H))

    out = pl.pallas_call(
        _body_b,
        grid=(B, K),
        in_specs=[
            bs((1, 1, NT, H), lambda b, k: (b, k, 0, 0)),
            bs((1, 1, NT, H), lambda b, k: (b, k, 0, 0)),
            bs((H, OUT), lambda b, k: (0, 0)),
            bs((1, OUT), lambda b, k: (0, 0)),
            bs((H, OUT), lambda b, k: (0, 0)),
            bs((1, OUT), lambda b, k: (0, 0)),
        ],
        out_specs=bs((1, NT, OUT), lambda b, k: (b, 0, 0)),
        out_shape=jax.ShapeDtypeStruct((B, NT, OUT), jnp.float32),
        scratch_shapes=[
            pltpu.VMEM((NT, OUT), jnp.float32),
            pltpu.VMEM((NT, OUT), jnp.float32),
        ],
    )(hf, hd, W_D2.T, b_D2[None, :], W_F2.T, b_F2[None, :])

    return jnp.transpose(jnp.reshape(out, (B, N, T, OUT)), (0, 3, 1, 2))


# speedup vs baseline: 6.7813x; 1.5310x over previous
import jax
import jax.numpy as jnp
from jax.experimental import pallas as pl

B, C, N, T, K, OUT = 2, 32, 128, 24, 8, 64
H = OUT // 2
BT = B * T
TC = T * C
BIG = 3.0e38


def _topk_onehots(dmat, k, rows):
    cols = jax.lax.broadcasted_iota(jnp.int32, (rows, N), 1)
    ohs = []
    d = dmat
    for _ in range(k):
        m = jnp.min(d, axis=1, keepdims=True)
        idx = jnp.min(jnp.where(d == m, cols, N), axis=1, keepdims=True)
        sel = cols == idx
        ohs.append(sel.astype(jnp.float32))
        d = jnp.where(sel, BIG, d)
    return ohs


def _dot3(x, w, prec):
    return jax.lax.dot_general(x, w, (((2,), (0,)), ((), ())), precision=prec)


def _bdot(oh, v, prec):
    return jax.lax.dot_general(oh, v, (((2,), (1,)), ((0,), (0,))),
                               precision=prec)


def _body(xg_ref, x3_ref, post_ref, dist_ref,
          wd1t_ref, bd1_ref, wd2t_ref, bd2_ref,
          wf1t_ref, bf1_ref, wf2t_ref, bf2_ref,
          out_ref):
    hi = jax.lax.Precision.HIGHEST
    x3 = x3_ref[...]
    post = post_ref[...]
    dist = dist_ref[...]

    ds = []
    for b in range(B):
        xgb = xg_ref[b]
        rn = jnp.sum(xgb * xgb, axis=1, keepdims=True)
        g = jax.lax.dot_general(xgb, xgb, (((1,), (1,)), ((), ())),
                                precision=hi)
        ds.append(rn + jnp.transpose(rn) - 2.0 * g)
    dall = jnp.concatenate(ds, axis=0)

    oh_f = _topk_onehots(dall, K, B * N)
    oh_d = _topk_onehots(dist, K, N)

    wf1t = wf1t_ref[...]
    wa = wf1t[:C, :]
    wb = wf1t[C:, :]
    wd1t = wd1t_ref[...]
    wx = wd1t[:C, :]
    wp = wd1t[C:C + 2, :]
    wdx = wd1t[C + 2:2 * C + 2, :]
    wdp = wd1t[2 * C + 2:, :]

    a_f = _dot3(x3, wa - wb, hi) + bf1_ref[...]
    b_f = _dot3(x3, wb, hi)
    posa = jnp.dot(post, wp - wdp, precision=hi)
    posb = jnp.dot(post, wdp, precision=hi)
    a_d = _dot3(x3, wx - wdx, hi) + posa + bd1_ref[...]
    b_d = _dot3(x3, wdx, hi) + posb

    wf2t = wf2t_ref[...]
    bf2 = bf2_ref[...]
    wd2t = wd2t_ref[...]
    bd2 = bd2_ref[...]

    accf = jnp.full((BT, N, OUT), -BIG, dtype=jnp.float32)
    accd = jnp.full((BT, N, OUT), -BIG, dtype=jnp.float32)
    for k in range(K):
        ohf3 = jnp.reshape(oh_f[k], (B, 1, N, N))
        ohf3 = jnp.reshape(jnp.broadcast_to(ohf3, (B, T, N, N)), (BT, N, N))
        pre = a_f + _bdot(ohf3, b_f, hi)
        h = jax.nn.relu(pre)
        o = jax.nn.relu(_dot3(h, wf2t, hi) + bf2)
        accf = jnp.maximum(accf, o)

        ohd3 = jnp.broadcast_to(oh_d[k][None], (BT, N, N))
        pre = a_d + _bdot(ohd3, b_d, hi)
        h = jax.nn.relu(pre)
        o = jax.nn.relu(_dot3(h, wd2t, hi) + bd2)
        accd = jnp.maximum(accd, o)

    out_ref[...] = accf + accd


def kernel(X, pos, dist, W_D1, b_D1, W_D2, b_D2, W_F1, b_F1, W_F2, b_F2):
    xg = jnp.reshape(jnp.transpose(X, (0, 2, 3, 1)), (B, N, TC))
    x3 = jnp.reshape(jnp.transpose(X, (0, 3, 2, 1)), (BT, N, C))

    bs = pl.BlockSpec
    out = pl.pallas_call(
        _body,
        in_specs=[
            bs((B, N, TC), lambda: (0, 0, 0)),
            bs((BT, N, C), lambda: (0, 0, 0)),
            bs((N, 2), lambda: (0, 0)),
            bs((N, N), lambda: (0, 0)),
            bs((2 * C + 4, H), lambda: (0, 0)),
            bs((1, H), lambda: (0, 0)),
            bs((H, OUT), lambda: (0, 0)),
            bs((1, OUT), lambda: (0, 0)),
            bs((2 * C, H), lambda: (0, 0)),
            bs((1, H), lambda: (0, 0)),
            bs((H, OUT), lambda: (0, 0)),
            bs((1, OUT), lambda: (0, 0)),
        ],
        out_specs=bs((BT, N, OUT), lambda: (0, 0, 0)),
        out_shape=jax.ShapeDtypeStruct((BT, N, OUT), jnp.float32),
    )(xg, x3, jnp.transpose(pos), dist,
      W_D1.T, b_D1[None, :], W_D2.T, b_D2[None, :],
      W_F1.T, b_F1[None, :], W_F2.T, b_F2[None, :])

    return jnp.transpose(jnp.reshape(out, (B, T, N, OUT)), (0, 3, 2, 1))


# speedup vs baseline: 6.7845x; 1.0005x over previous
import jax
import jax.numpy as jnp
from jax.experimental import pallas as pl

B, C, N, T, K, OUT = 2, 32, 128, 24, 8, 64
H = OUT // 2
BT = B * T
TC = T * C
BIG = 3.0e38


def _topk_onehots(dmat, k, rows):
    cols = jax.lax.broadcasted_iota(jnp.int32, (rows, N), 1)
    ohs = []
    d = dmat
    for _ in range(k):
        m = jnp.min(d, axis=1, keepdims=True)
        idx = jnp.min(jnp.where(d == m, cols, N), axis=1, keepdims=True)
        sel = cols == idx
        ohs.append(sel.astype(jnp.float32))
        d = jnp.where(sel, BIG, d)
    return ohs


def _dot3(x, w, prec):
    return jax.lax.dot_general(x, w, (((2,), (0,)), ((), ())), precision=prec)


def _bdot(oh, v, prec):
    return jax.lax.dot_general(oh, v, (((2,), (1,)), ((0,), (0,))),
                               precision=prec)


def _body(xg_ref, x3_ref, post_ref, dist_ref,
          wd1t_ref, bd1_ref, wd2t_ref, bd2_ref,
          wf1t_ref, bf1_ref, wf2t_ref, bf2_ref,
          out_ref):
    hi = jax.lax.Precision.HIGHEST
    md = jax.lax.Precision.HIGHEST
    x3 = x3_ref[...]
    post = post_ref[...]
    dist = dist_ref[...]

    ds = []
    for b in range(B):
        xgb = xg_ref[b]
        rn = jnp.sum(xgb * xgb, axis=1, keepdims=True)
        g = jax.lax.dot_general(xgb, xgb, (((1,), (1,)), ((), ())),
                                precision=hi)
        ds.append(rn + jnp.transpose(rn) - 2.0 * g)
    dall = jnp.concatenate(ds, axis=0)

    oh_f = _topk_onehots(dall, K, B * N)
    oh_d = _topk_onehots(dist, K, N)

    wf1t = wf1t_ref[...]
    wa = wf1t[:C, :]
    wb = wf1t[C:, :]
    wd1t = wd1t_ref[...]
    wx = wd1t[:C, :]
    wp = wd1t[C:C + 2, :]
    wdx = wd1t[C + 2:2 * C + 2, :]
    wdp = wd1t[2 * C + 2:, :]

    a_f = _dot3(x3, wa - wb, md) + bf1_ref[...]
    b_f = _dot3(x3, wb, md)
    posa = jnp.dot(post, wp - wdp, precision=md)
    posb = jnp.dot(post, wdp, precision=md)
    a_d = _dot3(x3, wx - wdx, md) + posa + bd1_ref[...]
    b_d = _dot3(x3, wdx, md) + posb

    wf2t = wf2t_ref[...]
    bf2 = bf2_ref[...]
    wd2t = wd2t_ref[...]
    bd2 = bd2_ref[...]

    accf = jnp.full((BT, N, OUT), -BIG, dtype=jnp.float32)
    accd = jnp.full((BT, N, OUT), -BIG, dtype=jnp.float32)
    for k in range(K):
        ohf3 = jnp.reshape(oh_f[k], (B, 1, N, N))
        ohf3 = jnp.reshape(jnp.broadcast_to(ohf3, (B, T, N, N)), (BT, N, N))
        pre = a_f + _bdot(ohf3, b_f, md)
        h = jax.nn.relu(pre)
        o = jax.nn.relu(_dot3(h, wf2t, md) + bf2)
        accf = jnp.maximum(accf, o)

        ohd3 = jnp.broadcast_to(oh_d[k][None], (BT, N, N))
        pre = a_d + _bdot(ohd3, b_d, md)
        h = jax.nn.relu(pre)
        o = jax.nn.relu(_dot3(h, wd2t, md) + bd2)
        accd = jnp.maximum(accd, o)

    out_ref[...] = accf + accd


def kernel(X, pos, dist, W_D1, b_D1, W_D2, b_D2, W_F1, b_F1, W_F2, b_F2):
    xg = jnp.reshape(jnp.transpose(X, (0, 2, 3, 1)), (B, N, TC))
    x3 = jnp.reshape(jnp.transpose(X, (0, 3, 2, 1)), (BT, N, C))

    bs = pl.BlockSpec
    out = pl.pallas_call(
        _body,
        in_specs=[
            bs((B, N, TC), lambda: (0, 0, 0)),
            bs((BT, N, C), lambda: (0, 0, 0)),
            bs((N, 2), lambda: (0, 0)),
            bs((N, N), lambda: (0, 0)),
            bs((2 * C + 4, H), lambda: (0, 0)),
            bs((1, H), lambda: (0, 0)),
            bs((H, OUT), lambda: (0, 0)),
            bs((1, OUT), lambda: (0, 0)),
            bs((2 * C, H), lambda: (0, 0)),
            bs((1, H), lambda: (0, 0)),
            bs((H, OUT), lambda: (0, 0)),
            bs((1, OUT), lambda: (0, 0)),
        ],
        out_specs=bs((BT, N, OUT), lambda: (0, 0, 0)),
        out_shape=jax.ShapeDtypeStruct((BT, N, OUT), jnp.float32),
    )(xg, x3, jnp.transpose(pos), dist,
      W_D1.T, b_D1[None, :], W_D2.T, b_D2[None, :],
      W_F1.T, b_F1[None, :], W_F2.T, b_F2[None, :])

    return jnp.transpose(jnp.reshape(out, (B, T, N, OUT)), (0, 3, 2, 1))
